# baseline (speedup 1.0000x reference)
import functools

import jax
import jax.numpy as jnp
from jax import lax
from jax.experimental import pallas as pl
from jax.experimental.pallas import tpu as pltpu
from jax.experimental.pallas import tpu_sc as plsc

_ALPHA = 0.1
_LANES = 16


def _sc_conv(feat2, f02, src, dst, weight, n, d):
    e = weight.shape[0]
    info = plsc.get_sparse_core_info()
    nc, ns = info.num_cores, info.num_subcores
    dh = d // nc

    per_sub = e // ns
    assert per_sub * ns == e
    n_phases = 2 if per_sub > 16384 else 1
    per_phase = per_sub // n_phases
    assert per_phase * n_phases == per_sub and per_phase % 8 == 0

    chunk = 128
    n_full = per_phase // chunk
    tail_e = per_phase - n_full * chunk
    assert tail_e % _LANES == 0

    rp = 8
    for c in range(80, 7, -8):
        if n % c == 0:
            rp = c
            break
    n_row_pieces = n // rp

    mesh = plsc.VectorSubcoreMesh(core_axis_name="c", subcore_axis_name="s")

    @functools.partial(
        pl.kernel,
        mesh=mesh,
        compiler_params=pltpu.CompilerParams(use_tc_tiling_on_sc=False),
        out_type=jax.ShapeDtypeStruct((nc * n, dh), jnp.float32),
        scratch_types=[
            pltpu.VMEM_SHARED((n, dh), jnp.float32),
            pltpu.VMEM((per_phase,), jnp.int32),
            pltpu.VMEM((per_phase,), jnp.int32),
            pltpu.VMEM((per_phase,), jnp.float32),
            pltpu.VMEM((chunk,), jnp.int32),
            pltpu.VMEM((chunk,), jnp.int32),
            pltpu.VMEM((rp,), jnp.int32),
            pltpu.VMEM((chunk, 64), jnp.float32),
            pltpu.VMEM((chunk, 64), jnp.float32),
            pltpu.SemaphoreType.DMA,
            pltpu.SemaphoreType.DMA,
            pltpu.SemaphoreType.DMA,
        ],
    )
    def sc_kernel(feat_hbm, f0_hbm, src_hbm, dst_hbm, w_hbm, out_hbm,
                  hi_sh, src_v, dst_v, w_v, idx_a, idx_b, idx_o,
                  rows_a, rows_b, sem_a, sem_b, sem_s):
        cid = lax.axis_index("c")
        sid = lax.axis_index("s")

        def zero_row(r, carry):
            for j in range(dh // _LANES):
                rows_a[r, pl.ds(j * _LANES, _LANES)] = jnp.zeros(
                    (_LANES,), jnp.float32)
            return carry
        lax.fori_loop(0, chunk, zero_row, 0)

        def zero_piece(k, carry):
            rc = sid + ns * k

            @pl.when(rc < n_row_pieces)
            def _():
                pltpu.sync_copy(rows_a.at[pl.ds(0, rp)],
                                hi_sh.at[pl.ds(rc * rp, rp)])
            return carry
        lax.fori_loop(0, (n_row_pieces + ns - 1) // ns, zero_piece, 0)
        plsc.subcore_barrier()

        def prep_idx(base, ibuf, size=chunk):
            def grp(g, carry):
                s16 = src_v[pl.ds(base + g * _LANES, _LANES)]
                ibuf[pl.ds(g * _LANES, _LANES)] = s16 * 2 + cid
                return carry
            lax.fori_loop(0, size // _LANES, grp, 0)

        def start_gather(ibuf, buf, sem, size=chunk):
            pltpu.async_copy(feat_hbm.at[ibuf.at[pl.ds(0, size)]],
                             buf.at[pl.ds(0, size)], sem)

        def wait_gather(ibuf, buf, sem, size=chunk):
            pltpu.make_async_copy(feat_hbm.at[ibuf.at[pl.ds(0, size)]],
                                  buf.at[pl.ds(0, size)], sem).wait()

        def process(base, buf, size=chunk):
            def scale_scatter_group(g, c2):
                wvec = w_v[pl.ds(base + g * _LANES, _LANES)]
                for i in range(_LANES):
                    r = g * _LANES + i
                    wspl = jnp.full((_LANES,), wvec[i], jnp.float32)
                    for j in range(dh // _LANES):
                        sl = pl.ds(j * _LANES, _LANES)
                        buf[r, sl] = buf[r, sl] * wspl
                dst16 = dst_v[pl.ds(base + g * _LANES, _LANES)]
                pltpu.async_copy(buf.at[pl.ds(g * _LANES, _LANES)],
                                 hi_sh.at[dst16], sem_s, add=True)
                return c2
            lax.fori_loop(0, size // _LANES, scale_scatter_group, 0)

            def drain_group(g, c2):
                dst16 = dst_v[pl.ds(base + g * _LANES, _LANES)]
                pltpu.make_async_copy(buf.at[pl.ds(g * _LANES, _LANES)],
                                      hi_sh.at[dst16], sem_s).wait()
                return c2
            lax.fori_loop(0, size // _LANES, drain_group, 0)

        for phase in range(n_phases):
            edge0 = sid * per_sub + phase * per_phase
            pltpu.sync_copy(src_hbm.at[pl.ds(edge0, per_phase)], src_v)
            pltpu.sync_copy(dst_hbm.at[pl.ds(edge0, per_phase)], dst_v)
            pltpu.sync_copy(w_hbm.at[pl.ds(edge0, per_phase)], w_v)

            prep_idx(0, idx_a)
            start_gather(idx_a, rows_a, sem_a)
            npairs = (n_full - 1) // 2

            def pair_body(k, carry):
                b0 = 2 * k * chunk
                b1 = b0 + chunk
                prep_idx(b1, idx_b)
                start_gather(idx_b, rows_b, sem_b)
                wait_gather(idx_a, rows_a, sem_a)
                process(b0, rows_a)
                prep_idx(b0 + 2 * chunk, idx_a)
                start_gather(idx_a, rows_a, sem_a)
                wait_gather(idx_b, rows_b, sem_b)
                process(b1, rows_b)
                return carry
            lax.fori_loop(0, npairs, pair_body, 0)

            t0 = 2 * npairs * chunk
            if n_full - 2 * npairs == 2:
                prep_idx(t0 + chunk, idx_b)
                start_gather(idx_b, rows_b, sem_b)
            wait_gather(idx_a, rows_a, sem_a)
            process(t0, rows_a)
            if n_full - 2 * npairs == 2:
                wait_gather(idx_b, rows_b, sem_b)
                process(t0 + chunk, rows_b)

            if tail_e:
                tb = n_full * chunk
                prep_idx(tb, idx_a, tail_e)
                start_gather(idx_a, rows_a, sem_a, tail_e)
                wait_gather(idx_a, rows_a, sem_a, tail_e)
                process(tb, rows_a, tail_e)

        plsc.subcore_barrier()

        lane = lax.iota(jnp.int32, _LANES)

        def blend_piece(k, carry):
            rc = sid + ns * k

            @pl.when(rc < n_row_pieces)
            def _():
                pltpu.sync_copy(hi_sh.at[pl.ds(rc * rp, rp)],
                                rows_a.at[pl.ds(0, rp)])
                for g in range(rp // _LANES):
                    r16 = (rc * rp + g * _LANES) + lane
                    idx_o[pl.ds(g * _LANES, _LANES)] = r16 * 2 + cid
                pltpu.async_copy(f0_hbm.at[idx_o],
                                 rows_b.at[pl.ds(0, rp)], sem_b).wait()

                def blend_group(g, c2):
                    for i in range(_LANES):
                        r = g * _LANES + i
                        for j in range(dh // _LANES):
                            sl = pl.ds(j * _LANES, _LANES)
                            x = (jnp.float32(1.0 - _ALPHA) * rows_a[r, sl]
                                 + jnp.float32(_ALPHA) * rows_b[r, sl])
                            rows_a[r, sl] = jnp.maximum(x, jnp.float32(0.0))
                    return c2
                lax.fori_loop(0, rp // _LANES, blend_group, 0)

                pltpu.sync_copy(rows_a.at[pl.ds(0, rp)],
                                out_hbm.at[idx_o])
            return carry
        lax.fori_loop(0, (n_row_pieces + ns - 1) // ns, blend_piece, 0)

    return sc_kernel(feat2, f02, src, dst, weight)


@jax.jit
def kernel(features, features0, edge_index, edge_weight):
    n, d = features.shape
    nc = plsc.get_sparse_core_info().num_cores
    dh = d // nc
    feat2 = features.reshape(nc * n, dh)
    f02 = features0.reshape(nc * n, dh)
    dst = edge_index[0]
    src = edge_index[1]
    out2 = _sc_conv(feat2, f02, src, dst, edge_weight, n, d)
    return out2.reshape(n, d)


# speedup vs baseline: 2.5276x; 2.5276x over previous
import functools

import jax
import jax.numpy as jnp
from jax import lax
from jax.experimental import pallas as pl
from jax.experimental.pallas import tpu as pltpu
from jax.experimental.pallas import tpu_sc as plsc

_ALPHA = 0.1
_LANES = 16


def _sc_partials(features, edge_index, weight):
    n, d = features.shape
    e = weight.shape[0]
    info = plsc.get_sparse_core_info()
    nc, ns = info.num_cores, info.num_subcores
    nw = nc * ns

    per_tile = e // nw
    assert per_tile * nw == e
    chunk = 16
    for c in range(128, 15, -16):
        if per_tile % c == 0:
            chunk = c
            break
    n_full = per_tile // chunk
    tail_e = per_tile - n_full * chunk
    assert tail_e % _LANES == 0

    rp = 8
    for c in range(80, 7, -8):
        if n % c == 0:
            rp = c
            break
    n_row_pieces = n // rp
    assert rp * n_row_pieces == n

    mesh = plsc.VectorSubcoreMesh(core_axis_name="c", subcore_axis_name="s")

    @functools.partial(
        pl.kernel,
        mesh=mesh,
        out_type=jax.ShapeDtypeStruct((nc, n, d), jnp.float32),
        scratch_types=[
            pltpu.VMEM_SHARED((n, d), jnp.float32),
            pltpu.VMEM((per_tile,), jnp.int32),
            pltpu.VMEM((per_tile,), jnp.int32),
            pltpu.VMEM((per_tile,), jnp.float32),
            pltpu.VMEM((chunk, d), jnp.float32),
            pltpu.VMEM((chunk, d), jnp.float32),
            pltpu.SemaphoreType.DMA,
            pltpu.SemaphoreType.DMA,
            pltpu.SemaphoreType.DMA,
        ],
    )
    def sc_kernel(feat_hbm, ei_hbm, w_hbm, out_hbm,
                  hi_sh, src_v, dst_v, w_v, rows_a, rows_b,
                  sem_a, sem_b, sem_s):
        rows_v = rows_a
        cid = lax.axis_index("c")
        sid = lax.axis_index("s")
        wid = sid * nc + cid

        edge0 = wid * per_tile
        stage = [
            (ei_hbm.at[pl.ds(e + edge0, per_tile)], src_v),
            (ei_hbm.at[pl.ds(edge0, per_tile)], dst_v),
            (w_hbm.at[pl.ds(edge0, per_tile)], w_v),
        ]
        for s_src, s_dst in stage:
            pltpu.async_copy(s_src, s_dst, sem_b)

        def zero_row(r, carry):
            for j in range(d // _LANES):
                rows_v[r, pl.ds(j * _LANES, _LANES)] = jnp.zeros(
                    (_LANES,), jnp.float32)
            return carry
        lax.fori_loop(0, chunk, zero_row, 0)

        def zero_chunk(k, carry):
            rc = sid + ns * k

            @pl.when(rc < n_row_pieces)
            def _():
                pltpu.async_copy(rows_v.at[pl.ds(0, rp)],
                                 hi_sh.at[pl.ds(rc * rp, rp)], sem_a)
            return carry
        lax.fori_loop(0, (n_row_pieces + ns - 1) // ns, zero_chunk, 0)

        def zero_drain(k, carry):
            rc = sid + ns * k

            @pl.when(rc < n_row_pieces)
            def _():
                pltpu.make_async_copy(
                    rows_v.at[pl.ds(0, rp)],
                    hi_sh.at[pl.ds(rc * rp, rp)], sem_a).wait()
            return carry
        lax.fori_loop(0, (n_row_pieces + ns - 1) // ns, zero_drain, 0)
        for s_src, s_dst in stage:
            pltpu.make_async_copy(s_src, s_dst, sem_b).wait()
        plsc.subcore_barrier()

        def start_gather(base, buf, sem, size=chunk):
            pltpu.async_copy(
                feat_hbm.at[src_v.at[pl.ds(base, size)]],
                buf.at[pl.ds(0, size)], sem)

        def wait_gather(base, buf, sem, size=chunk):
            pltpu.make_async_copy(
                feat_hbm.at[src_v.at[pl.ds(base, size)]],
                buf.at[pl.ds(0, size)], sem).wait()

        def process(base, buf, size=chunk):
            def scale_scatter_group(g, c2):
                wvec = w_v[pl.ds(base + g * _LANES, _LANES)]
                for i in range(_LANES):
                    r = g * _LANES + i
                    wspl = jnp.full((_LANES,), wvec[i], jnp.float32)
                    for j in range(d // _LANES):
                        sl = pl.ds(j * _LANES, _LANES)
                        buf[r, sl] = buf[r, sl] * wspl
                dst16 = dst_v[pl.ds(base + g * _LANES, _LANES)]
                pltpu.async_copy(buf.at[pl.ds(g * _LANES, _LANES)],
                                 hi_sh.at[dst16], sem_s, add=True)
                return c2
            lax.fori_loop(0, size // _LANES, scale_scatter_group, 0)

            def drain_group(g, c2):
                dst16 = dst_v[pl.ds(base + g * _LANES, _LANES)]
                pltpu.make_async_copy(buf.at[pl.ds(g * _LANES, _LANES)],
                                      hi_sh.at[dst16], sem_s).wait()
                return c2
            lax.fori_loop(0, size // _LANES, drain_group, 0)

        start_gather(0, rows_a, sem_a)
        npairs = (n_full - 1) // 2

        def pair_body(k, carry):
            b0 = 2 * k * chunk
            b1 = b0 + chunk
            start_gather(b1, rows_b, sem_b)
            wait_gather(b0, rows_a, sem_a)
            process(b0, rows_a)
            start_gather(b0 + 2 * chunk, rows_a, sem_a)
            wait_gather(b1, rows_b, sem_b)
            process(b1, rows_b)
            return carry
        lax.fori_loop(0, npairs, pair_body, 0)

        t0 = 2 * npairs * chunk
        if n_full - 2 * npairs == 2:
            start_gather(t0 + chunk, rows_b, sem_b)
        wait_gather(t0, rows_a, sem_a)
        process(t0, rows_a)
        if n_full - 2 * npairs == 2:
            wait_gather(t0 + chunk, rows_b, sem_b)
            process(t0 + chunk, rows_b)

        if tail_e:
            tb = n_full * chunk
            start_gather(tb, rows_a, sem_a, tail_e)
            wait_gather(tb, rows_a, sem_a, tail_e)
            process(tb, rows_a, tail_e)

        plsc.subcore_barrier()

        def out_chunk(k, carry):
            rc = sid + ns * k

            @pl.when(rc < n_row_pieces)
            def _():
                pltpu.async_copy(hi_sh.at[pl.ds(rc * rp, rp)],
                                 out_hbm.at[cid, pl.ds(rc * rp, rp)], sem_a)
            return carry
        lax.fori_loop(0, (n_row_pieces + ns - 1) // ns, out_chunk, 0)

        def out_drain(k, carry):
            rc = sid + ns * k

            @pl.when(rc < n_row_pieces)
            def _():
                pltpu.make_async_copy(
                    hi_sh.at[pl.ds(rc * rp, rp)],
                    out_hbm.at[cid, pl.ds(rc * rp, rp)], sem_a).wait()
            return carry
        lax.fori_loop(0, (n_row_pieces + ns - 1) // ns, out_drain, 0)

    return sc_kernel(features, edge_index.reshape(2 * e), weight)


def _combine(p0, p1, features0):
    n, d = features0.shape
    blk = 2000
    assert n % blk == 0

    def body(p0_ref, p1_ref, f0_ref, o_ref):
        hi = p0_ref[...] + p1_ref[...]
        x = jnp.float32(1.0 - _ALPHA) * hi + jnp.float32(_ALPHA) * f0_ref[...]
        o_ref[...] = jnp.maximum(x, jnp.float32(0.0))

    spec = pl.BlockSpec((blk, d), lambda i: (i, 0))
    return pl.pallas_call(
        body,
        grid=(n // blk,),
        in_specs=[spec, spec, spec],
        out_specs=spec,
        out_shape=jax.ShapeDtypeStruct((n, d), jnp.float32),
    )(p0, p1, features0)


@jax.jit
def kernel(features, features0, edge_index, edge_weight):
    partials = _sc_partials(features, edge_index, edge_weight)
    return _combine(partials[0], partials[1], features0)
